# baseline (speedup 1.0000x reference)
import functools

import jax
import jax.numpy as jnp
from jax import lax
from jax.experimental import pallas as pl
from jax.experimental.pallas import tpu as pltpu
from jax.experimental.pallas import tpu_sc as plsc

B, S, D, K = 4, 4096, 1024, 512
L = 16
NB = S // L
ROWS_PER_TILE = K // 8
CPAD = K + L
SPAD = K + 2 * L


def _score_body(t_ref, w_ref, s_ref, meta_ref):
    t2 = t_ref[...].reshape(S, D)
    s = lax.dot_general(w_ref[...], t2, (((1,), (1,)), ((), ())),
                        preferred_element_type=jnp.float32)
    s_ref[...] = s.reshape(1, 1, S)
    bi = lax.bitcast_convert_type(s, jnp.int32)
    key = bi ^ ((bi >> 31) & jnp.int32(0x7FFFFFFF))
    u = lax.bitcast_convert_type(key, jnp.uint32) ^ jnp.uint32(0x80000000)
    t = jnp.uint32(0)
    for step in range(32):
        trial = t | jnp.uint32(1 << (31 - step))
        cnt = jnp.sum((u >= trial).astype(jnp.int32))
        t = jnp.where(cnt >= K, trial, t)
    count_gt = jnp.sum((u > t).astype(jnp.int32))
    m = jnp.int32(K) - count_gt
    t_i = lax.bitcast_convert_type(t, jnp.int32)
    io = lax.broadcasted_iota(jnp.int32, (1, 1, 128), 2)
    meta_ref[...] = jnp.where(
        io == 0, t_i, jnp.where(io == 1, count_gt,
                                jnp.where(io == 2, m, jnp.int32(0))))


def _make_scores_tc(pair):
    def call(token, w_row):
        return pl.pallas_call(
            _score_body,
            grid=(2,),
            in_specs=[
                pl.BlockSpec((1, S, D), lambda b: (2 * pair + b, 0, 0)),
                pl.BlockSpec((1, D), lambda b: (0, 0)),
            ],
            out_specs=[
                pl.BlockSpec((1, 1, S), lambda b: (b, 0, 0)),
                pl.BlockSpec((1, 1, 128), lambda b: (b, 0, 0)),
            ],
            out_shape=[
                jax.ShapeDtypeStruct((2, 1, S), jnp.float32),
                jax.ShapeDtypeStruct((2, 1, 128), jnp.int32),
            ],
        )(token, w_row)
    return call


_mesh = plsc.VectorSubcoreMesh(core_axis_name="c", subcore_axis_name="s")
RPT2 = K // 16


def _make_sc(pair):
    @functools.partial(
        pl.kernel,
        mesh=_mesh,
        compiler_params=pltpu.CompilerParams(needs_layout_passes=False),
        out_type=jax.ShapeDtypeStruct((2 * K, D), jnp.float32),
        scratch_types=[
            pltpu.VMEM((S,), jnp.float32),
            pltpu.VMEM((S,), jnp.uint32),
            pltpu.VMEM((128,), jnp.int32),
            pltpu.VMEM((CPAD,), jnp.uint32),
            pltpu.VMEM((CPAD,), jnp.int32),
            pltpu.VMEM((S + L,), jnp.int32),
            pltpu.VMEM((2 * L,), jnp.int32),
            pltpu.VMEM((16,), jnp.int32),
            pltpu.VMEM((4 * L,), jnp.int32),
            pltpu.VMEM((4 * L,), jnp.int32),
            pltpu.VMEM((RPT2,), jnp.int32),
            pltpu.VMEM((RPT2, D), jnp.float32),
            pltpu.VMEM_SHARED((CPAD,), jnp.uint32),
            pltpu.VMEM_SHARED((CPAD,), jnp.int32),
            pltpu.VMEM_SHARED((CPAD,), jnp.int32),
            pltpu.VMEM_SHARED((16,), jnp.int32),
            pltpu.VMEM_SHARED((SPAD,), jnp.int32),
            pltpu.SemaphoreType.DMA,
        ],
    )
    def _sc(scores_hbm, meta_hbm, token_hbm, out_hbm,
            scf_v, u_v, meta128_v, cu_v, cidx_v, tied_v, tloc_v,
            meta_v, rank_v, rowid_v, idx_v, rows_v,
            sh_cu, sh_cidx, sh_tied, sh_meta, sh_sorted, sem):
        cid = lax.axis_index("c")
        sid = lax.axis_index("s")
        iota = lax.iota(jnp.int32, L)
        base = (2 * pair + cid) * jnp.int32(S)

        @pl.when(sid == 0)
        def _phase1a():
            pltpu.sync_copy(scores_hbm.at[cid], scf_v)

            def xform(i, carry):
                f = scf_v[pl.ds(i * L, L)]
                bi = lax.bitcast_convert_type(f, jnp.int32)
                key = bi ^ ((bi >> 31) & jnp.int32(0x7FFFFFFF))
                u_v[pl.ds(i * L, L)] = (
                    lax.bitcast_convert_type(key, jnp.uint32)
                    ^ jnp.uint32(0x80000000))
                return carry
            lax.fori_loop(0, NB, xform, 0)

            pltpu.sync_copy(meta_hbm.at[cid], meta128_v)
            mv = meta128_v[pl.ds(0, L)]
            t = lax.bitcast_convert_type(mv, jnp.uint32)[0]
            count_gt = mv[1]
            m = mv[2]

            def zfill(i, carry):
                cu_v[pl.ds(i * L, L)] = jnp.zeros((L,), jnp.uint32)
                return carry
            lax.fori_loop(0, CPAD // L, zfill, 0)

            def compact_body(i, carry):
                og, oe = carry
                x = u_v[pl.ds(i * L, L)]
                idxv = i * L + iota
                gt = x > t
                eq = x == t
                plsc.store_compressed(cu_v.at[pl.ds(og, L)], x, mask=gt)
                plsc.store_compressed(cidx_v.at[pl.ds(og, L)], idxv, mask=gt)
                plsc.store_compressed(tied_v.at[pl.ds(oe, L)], idxv, mask=eq)
                return (og + plsc.all_reduce_population_count(gt)[0],
                        oe + plsc.all_reduce_population_count(eq)[0])
            lax.fori_loop(0, NB, compact_body, (jnp.int32(0), jnp.int32(0)))

            meta_v[pl.ds(0, L)] = jnp.where(
                iota == 0, count_gt, jnp.where(iota == 1, m, jnp.int32(0)))
            pltpu.sync_copy(cu_v, sh_cu)
            pltpu.sync_copy(cidx_v, sh_cidx)
            pltpu.sync_copy(tied_v.at[pl.ds(0, CPAD)], sh_tied)
            pltpu.sync_copy(meta_v, sh_meta)

        plsc.subcore_barrier()

        pltpu.sync_copy(sh_cu, cu_v)
        pltpu.sync_copy(sh_cidx, cidx_v)
        pltpu.sync_copy(sh_tied.at[pl.ds(sid * (2 * L), 2 * L)], tloc_v)
        pltpu.sync_copy(sh_meta, meta_v)
        mvec = meta_v[pl.ds(0, L)]
        count_gt = mvec[0]
        m = mvec[1]
        nG = (count_gt + (L - 1)) // L

        for gi in range(2):
            gg = sid * 2 + gi
            iv = cu_v[pl.ds(gg * L, L)]
            iidx = cidx_v[pl.ds(gg * L, L)]

            def rank_inner(jv, r, iv=iv, iidx=iidx):
                uj16 = cu_v[pl.ds(jv * L, L)]
                ij16 = cidx_v[pl.ds(jv * L, L)]
                for lane in range(L):
                    uj = uj16[lane]
                    ij = ij16[lane]
                    hit = (uj > iv) | ((uj == iv) & (ij < iidx))
                    r = r + hit.astype(jnp.int32)
                return r
            r = lax.fori_loop(0, nG, rank_inner, jnp.zeros((L,), jnp.int32))
            lane_ok = (gg * L + iota) < count_gt
            rank_v[pl.ds(gi * L, L)] = jnp.where(
                lane_ok, r, jnp.int32(K) + iota)
            rowid_v[pl.ds(gi * L, L)] = iidx + base

        for tv in range(2):
            pp = sid * (2 * L) + tv * L + iota
            ti = tloc_v[pl.ds(tv * L, L)]
            lane_ok = pp < m
            rank_v[pl.ds(2 * L + tv * L, L)] = jnp.where(
                lane_ok, count_gt + pp, jnp.int32(K) + iota)
            rowid_v[pl.ds(2 * L + tv * L, L)] = ti + base

        pltpu.sync_copy(rowid_v, sh_sorted.at[rank_v])

        plsc.subcore_barrier()

        pltpu.sync_copy(sh_sorted.at[pl.ds(sid * RPT2, RPT2)], idx_v)
        pltpu.async_copy(token_hbm.at[idx_v], rows_v, sem).wait()
        row0 = cid * K + sid * RPT2
        pltpu.sync_copy(rows_v, out_hbm.at[pl.ds(row0, RPT2)])

    return _sc


_sc_pair = [_make_sc(0), _make_sc(1)]
_tc_pair = [_make_scores_tc(0), _make_scores_tc(1)]


def kernel(token, mask, label, w_att):
    w_row = w_att.reshape(1, D)
    token2 = token.reshape(B * S, D)
    outs = []
    for pair in range(2):
        scores3, meta3 = _tc_pair[pair](token, w_row)
        outs.append(_sc_pair[pair](
            scores3.reshape(2, S), meta3.reshape(2, 128), token2))
    return jnp.concatenate(outs, axis=0).reshape(B, K, D)


# speedup vs baseline: 1.1613x; 1.1613x over previous
import functools

import jax
import jax.numpy as jnp
from jax import lax
from jax.experimental import pallas as pl
from jax.experimental.pallas import tpu as pltpu
from jax.experimental.pallas import tpu_sc as plsc

B, S, D, K = 4, 4096, 1024, 512
L = 16
NB = S // L
ROWS_PER_TILE = K // 8
CPAD = K + L
SPAD = K + 2 * L


def _score_body(t_ref, w_ref, s_ref, meta_ref):
    t2 = t_ref[...].reshape(S, D)
    s = lax.dot_general(w_ref[...], t2, (((1,), (1,)), ((), ())),
                        preferred_element_type=jnp.float32)
    s_ref[...] = s.reshape(1, 1, S)
    bi = lax.bitcast_convert_type(s, jnp.int32)
    key = bi ^ ((bi >> 31) & jnp.int32(0x7FFFFFFF))
    u = lax.bitcast_convert_type(key, jnp.uint32) ^ jnp.uint32(0x80000000)
    t = jnp.uint32(0)
    for step in range(32):
        trial = t | jnp.uint32(1 << (31 - step))
        cnt = jnp.sum((u >= trial).astype(jnp.int32))
        t = jnp.where(cnt >= K, trial, t)
    count_gt = jnp.sum((u > t).astype(jnp.int32))
    m = jnp.int32(K) - count_gt
    t_i = lax.bitcast_convert_type(t, jnp.int32)
    io = lax.broadcasted_iota(jnp.int32, (1, 1, 128), 2)
    meta_ref[...] = jnp.where(
        io == 0, t_i, jnp.where(io == 1, count_gt,
                                jnp.where(io == 2, m, jnp.int32(0))))


def _scores_tc(token, w_row):
    return pl.pallas_call(
        _score_body,
        grid=(B,),
        in_specs=[
            pl.BlockSpec((1, S, D), lambda b: (b, 0, 0)),
            pl.BlockSpec((1, D), lambda b: (0, 0)),
        ],
        out_specs=[
            pl.BlockSpec((1, 1, S), lambda b: (b, 0, 0)),
            pl.BlockSpec((1, 1, 128), lambda b: (b, 0, 0)),
        ],
        out_shape=[
            jax.ShapeDtypeStruct((B, 1, S), jnp.float32),
            jax.ShapeDtypeStruct((B, 1, 128), jnp.int32),
        ],
    )(token, w_row)


_mesh = plsc.VectorSubcoreMesh(core_axis_name="c", subcore_axis_name="s")


@functools.partial(
    pl.kernel,
    mesh=_mesh,
    compiler_params=pltpu.CompilerParams(needs_layout_passes=False),
    out_type=jax.ShapeDtypeStruct((B * K, D), jnp.float32),
    scratch_types=[
        pltpu.VMEM((S,), jnp.float32),
        pltpu.VMEM((S,), jnp.uint32),
        pltpu.VMEM((128,), jnp.int32),
        pltpu.VMEM((CPAD,), jnp.uint32),
        pltpu.VMEM((CPAD,), jnp.int32),
        pltpu.VMEM((S + L,), jnp.int32),
        pltpu.VMEM((ROWS_PER_TILE,), jnp.int32),
        pltpu.VMEM((16,), jnp.int32),
        pltpu.VMEM((2 * ROWS_PER_TILE,), jnp.int32),
        pltpu.VMEM((2 * ROWS_PER_TILE,), jnp.int32),
        pltpu.VMEM((ROWS_PER_TILE,), jnp.int32),
        pltpu.VMEM((ROWS_PER_TILE, D), jnp.float32),
        pltpu.VMEM_SHARED((2 * CPAD,), jnp.uint32),
        pltpu.VMEM_SHARED((2 * CPAD,), jnp.int32),
        pltpu.VMEM_SHARED((2 * CPAD,), jnp.int32),
        pltpu.VMEM_SHARED((2 * 16,), jnp.int32),
        pltpu.VMEM_SHARED((2 * SPAD,), jnp.int32),
        pltpu.SemaphoreType.DMA,
    ],
)
def _sc_topk_gather(scores_hbm, meta_hbm, token_hbm, out_hbm,
                    scf_v, u_v, meta128_v, cu_v, cidx_v, tied_v, tloc_v,
                    meta_v, rank_v, rowid_v, idx_v, rows_v,
                    sh_cu, sh_cidx, sh_tied, sh_meta, sh_sorted, sem):
    cid = lax.axis_index("c")
    sid = lax.axis_index("s")
    iota = lax.iota(jnp.int32, L)

    @pl.when(sid < 2)
    def _phase1a():
        b = 2 * cid + sid
        pltpu.sync_copy(scores_hbm.at[b], scf_v)

        def xform(i, carry):
            f = scf_v[pl.ds(i * L, L)]
            bi = lax.bitcast_convert_type(f, jnp.int32)
            key = bi ^ ((bi >> 31) & jnp.int32(0x7FFFFFFF))
            u_v[pl.ds(i * L, L)] = (
                lax.bitcast_convert_type(key, jnp.uint32) ^ jnp.uint32(0x80000000))
            return carry
        lax.fori_loop(0, NB, xform, 0)

        pltpu.sync_copy(meta_hbm.at[b], meta128_v)
        mv = meta128_v[pl.ds(0, L)]
        t = lax.bitcast_convert_type(mv, jnp.uint32)[0]
        count_gt = mv[1]
        m = mv[2]

        def zfill(i, carry):
            cu_v[pl.ds(i * L, L)] = jnp.zeros((L,), jnp.uint32)
            return carry
        lax.fori_loop(0, CPAD // L, zfill, 0)

        def compact_body(i, carry):
            og, oe = carry
            x = u_v[pl.ds(i * L, L)]
            idxv = i * L + iota
            gt = x > t
            eq = x == t
            plsc.store_compressed(cu_v.at[pl.ds(og, L)], x, mask=gt)
            plsc.store_compressed(cidx_v.at[pl.ds(og, L)], idxv, mask=gt)
            plsc.store_compressed(tied_v.at[pl.ds(oe, L)], idxv, mask=eq)
            return (og + plsc.all_reduce_population_count(gt)[0],
                    oe + plsc.all_reduce_population_count(eq)[0])
        lax.fori_loop(0, NB, compact_body, (jnp.int32(0), jnp.int32(0)))

        meta_v[pl.ds(0, L)] = jnp.where(
            iota == 0, count_gt, jnp.where(iota == 1, m, jnp.int32(0)))
        pltpu.sync_copy(cu_v, sh_cu.at[pl.ds(sid * CPAD, CPAD)])
        pltpu.sync_copy(cidx_v, sh_cidx.at[pl.ds(sid * CPAD, CPAD)])
        pltpu.sync_copy(tied_v.at[pl.ds(0, CPAD)],
                        sh_tied.at[pl.ds(sid * CPAD, CPAD)])
        pltpu.sync_copy(meta_v, sh_meta.at[pl.ds(sid * 16, 16)])

    plsc.subcore_barrier()

    b1 = sid // 8
    chunk = sid % 8
    base = (2 * cid + b1) * jnp.int32(S)
    slab = b1 * jnp.int32(SPAD)

    pltpu.sync_copy(sh_cu.at[pl.ds(b1 * CPAD, CPAD)], cu_v)
    pltpu.sync_copy(sh_cidx.at[pl.ds(b1 * CPAD, CPAD)], cidx_v)
    pltpu.sync_copy(
        sh_tied.at[pl.ds(b1 * CPAD + chunk * ROWS_PER_TILE, ROWS_PER_TILE)],
        tloc_v)
    pltpu.sync_copy(sh_meta.at[pl.ds(b1 * 16, 16)], meta_v)
    mvec = meta_v[pl.ds(0, L)]
    count_gt = mvec[0]
    m = mvec[1]
    nG = (count_gt + (L - 1)) // L

    for gi in range(4):
        gg = chunk * 4 + gi
        iv = cu_v[pl.ds(gg * L, L)]
        iidx = cidx_v[pl.ds(gg * L, L)]

        def rank_inner(jv, r, iv=iv, iidx=iidx):
            uj16 = cu_v[pl.ds(jv * L, L)]
            ij16 = cidx_v[pl.ds(jv * L, L)]
            for lane in range(L):
                uj = uj16[lane]
                ij = ij16[lane]
                hit = (uj > iv) | ((uj == iv) & (ij < iidx))
                r = r + hit.astype(jnp.int32)
            return r
        r = lax.fori_loop(0, nG, rank_inner, jnp.zeros((L,), jnp.int32))
        lane_ok = (gg * L + iota) < count_gt
        rank_v[pl.ds(gi * L, L)] = slab + jnp.where(
            lane_ok, r, jnp.int32(K) + iota)
        rowid_v[pl.ds(gi * L, L)] = iidx + base

    for tv in range(4):
        p = chunk * ROWS_PER_TILE + tv * L + iota
        ti = tloc_v[pl.ds(tv * L, L)]
        lane_ok = p < m
        rank_v[pl.ds(ROWS_PER_TILE + tv * L, L)] = slab + jnp.where(
            lane_ok, count_gt + p, jnp.int32(K) + iota)
        rowid_v[pl.ds(ROWS_PER_TILE + tv * L, L)] = ti + base

    pltpu.sync_copy(rowid_v, sh_sorted.at[rank_v])

    plsc.subcore_barrier()

    pltpu.sync_copy(
        sh_sorted.at[pl.ds(slab + chunk * ROWS_PER_TILE, ROWS_PER_TILE)], idx_v)
    pltpu.async_copy(token_hbm.at[idx_v], rows_v, sem).wait()
    row0 = (2 * cid + b1) * K + chunk * ROWS_PER_TILE
    pltpu.sync_copy(rows_v, out_hbm.at[pl.ds(row0, ROWS_PER_TILE)])


def kernel(token, mask, label, w_att):
    scores3, meta3 = _scores_tc(token, w_att.reshape(1, D))
    token2 = token.reshape(B * S, D)
    out2 = _sc_topk_gather(scores3.reshape(B, S), meta3.reshape(B, 128), token2)
    return out2.reshape(B, K, D)
